# Initial kernel scaffold; baseline (speedup 1.0000x reference)
import jax
import jax.numpy as jnp
from jax.experimental import pallas as pl


def kernel(x, adj_indices, adj_values, W, b):
    raise NotImplementedError("write your pallas kernel here")


# speedup vs baseline: 4.6645x; 4.6645x over previous
import functools

import jax
import jax.numpy as jnp
from jax import lax
from jax.experimental import pallas as pl
from jax.experimental.pallas import tpu as pltpu
from jax.experimental.pallas import tpu_sc as plsc

N = 10000
E = 320000
D = 128
LANES = 16
CHUNK = 128
NCHUNKS = E // CHUNK
NC = 2
NS = 16
NW = NC * NS
ITERS = -(-NCHUNKS // NW)
RBLK = 80
NRBLK = N // RBLK
RITERS = -(-NRBLK // NS)


def _mm_body(x_ref, w_ref, b_ref, o_ref):
    o_ref[...] = (
        jnp.dot(x_ref[...], w_ref[...], preferred_element_type=jnp.float32)
        + b_ref[...]
    )


def _matmul_bias(x, W, b):
    M = x.shape[0]
    BM = 1000
    return pl.pallas_call(
        _mm_body,
        grid=(M // BM,),
        in_specs=[
            pl.BlockSpec((BM, D), lambda i: (i, 0)),
            pl.BlockSpec((D, D), lambda i: (0, 0)),
            pl.BlockSpec((1, D), lambda i: (0, 0)),
        ],
        out_specs=pl.BlockSpec((BM, D), lambda i: (i, 0)),
        out_shape=jax.ShapeDtypeStruct((M, D), jnp.float32),
    )(x, W, b.reshape(1, D))


def _add_body(a_ref, b_ref, o_ref):
    o_ref[...] = a_ref[...] + b_ref[...]


def _add2(a, b):
    BM = 1000
    return pl.pallas_call(
        _add_body,
        grid=(N // BM,),
        in_specs=[pl.BlockSpec((BM, D), lambda i: (i, 0))] * 2,
        out_specs=pl.BlockSpec((BM, D), lambda i: (i, 0)),
        out_shape=jax.ShapeDtypeStruct((N, D), jnp.float32),
    )(a, b)


def _sc_scatter(h, rows, cols, vals):
    mesh = plsc.VectorSubcoreMesh(core_axis_name="c", subcore_axis_name="s")

    @functools.partial(
        pl.kernel,
        out_type=jax.ShapeDtypeStruct((NC, N, D), jnp.float32),
        mesh=mesh,
        compiler_params=pltpu.CompilerParams(needs_layout_passes=False),
        scratch_types=[
            pltpu.VMEM((CHUNK,), jnp.int32),
            pltpu.VMEM((CHUNK,), jnp.int32),
            pltpu.VMEM((CHUNK,), jnp.float32),
            pltpu.VMEM((CHUNK, D), jnp.float32),
            pltpu.VMEM_SHARED((N, D), jnp.float32),
            pltpu.SemaphoreType.DMA,
        ],
    )
    def k(h_hbm, rows_hbm, cols_hbm, vals_hbm, out_hbm,
          cols_v, rows_v, vals_v, msgs_v, acc, sem):
        cid = lax.axis_index("c")
        sid = lax.axis_index("s")
        w = sid * NC + cid

        def zero_body(i, carry):
            for j in range(D // LANES):
                msgs_v[i, pl.ds(j * LANES, LANES)] = jnp.zeros(
                    (LANES,), jnp.float32)
            return carry

        lax.fori_loop(0, CHUNK, zero_body, 0)
        for t in range(RITERS):
            rb = sid + t * NS

            @pl.when(rb < NRBLK)
            def _():
                r0 = pl.multiple_of(rb * RBLK, 8)
                pltpu.sync_copy(
                    msgs_v.at[pl.ds(0, RBLK)],
                    acc.at[pl.ds(r0, RBLK)],
                )

        plsc.subcore_barrier()

        def chunk_body(i, carry):
            c = w + i * NW

            @pl.when(c < NCHUNKS)
            def _():
                off = pl.multiple_of(c * CHUNK, CHUNK)
                pltpu.sync_copy(cols_hbm.at[pl.ds(off, CHUNK)], cols_v)
                pltpu.sync_copy(rows_hbm.at[pl.ds(off, CHUNK)], rows_v)
                pltpu.sync_copy(vals_hbm.at[pl.ds(off, CHUNK)], vals_v)
                pltpu.async_copy(h_hbm.at[cols_v], msgs_v, sem).wait()

                def scale_body(e, carry2):
                    v = plsc.load_gather(
                        vals_v, [jnp.full((LANES,), e, jnp.int32)])
                    for j in range(D // LANES):
                        sl = msgs_v[e, pl.ds(j * LANES, LANES)]
                        msgs_v[e, pl.ds(j * LANES, LANES)] = sl * v
                    return carry2

                lax.fori_loop(0, CHUNK, scale_body, 0)
                pltpu.sync_copy(msgs_v, acc.at[rows_v], add=True)

            return carry

        lax.fori_loop(0, ITERS, chunk_body, 0)
        plsc.subcore_barrier()

        for t in range(RITERS):
            rb = sid + t * NS

            @pl.when(rb < NRBLK)
            def _():
                r0 = pl.multiple_of(rb * RBLK, 8)
                pltpu.sync_copy(
                    acc.at[pl.ds(r0, RBLK)],
                    out_hbm.at[cid, pl.ds(r0, RBLK)],
                )

    return k(h, rows, cols, vals)


def kernel(x, adj_indices, adj_values, W, b):
    h = _matmul_bias(x, W, b)
    parts = _sc_scatter(h, adj_indices[0], adj_indices[1], adj_values)
    return _add2(parts[0], parts[1])


# speedup vs baseline: 5.7830x; 1.2398x over previous
import functools

import jax
import jax.numpy as jnp
from jax import lax
from jax.experimental import pallas as pl
from jax.experimental.pallas import tpu as pltpu
from jax.experimental.pallas import tpu_sc as plsc

N = 10000
E = 320000
D = 128
LANES = 16
CHUNK = 128
NCHUNKS = E // CHUNK
NC = 2
NS = 16
NW = NC * NS
ITERS = -(-NCHUNKS // NW)
RBLK = 80
NRBLK = N // RBLK
RITERS = -(-NRBLK // NS)


def _mm_body(x_ref, w_ref, b_ref, o_ref):
    o_ref[...] = (
        jnp.dot(x_ref[...], w_ref[...], preferred_element_type=jnp.float32)
        + b_ref[...]
    )


def _matmul_bias(x, W, b):
    M = x.shape[0]
    BM = 1000
    return pl.pallas_call(
        _mm_body,
        grid=(M // BM,),
        in_specs=[
            pl.BlockSpec((BM, D), lambda i: (i, 0)),
            pl.BlockSpec((D, D), lambda i: (0, 0)),
            pl.BlockSpec((1, D), lambda i: (0, 0)),
        ],
        out_specs=pl.BlockSpec((BM, D), lambda i: (i, 0)),
        out_shape=jax.ShapeDtypeStruct((M, D), jnp.float32),
    )(x, W, b.reshape(1, D))


def _add_body(a_ref, b_ref, o_ref):
    o_ref[...] = a_ref[...] + b_ref[...]


def _add2(a, b):
    BM = 1000
    return pl.pallas_call(
        _add_body,
        grid=(N // BM,),
        in_specs=[pl.BlockSpec((BM, D), lambda i: (i, 0))] * 2,
        out_specs=pl.BlockSpec((BM, D), lambda i: (i, 0)),
        out_shape=jax.ShapeDtypeStruct((N, D), jnp.float32),
    )(a, b)


def _sc_scatter(h, edata):
    mesh = plsc.VectorSubcoreMesh(core_axis_name="c", subcore_axis_name="s")

    @functools.partial(
        pl.kernel,
        out_type=jax.ShapeDtypeStruct((NC, N, D), jnp.float32),
        mesh=mesh,
        compiler_params=pltpu.CompilerParams(needs_layout_passes=False),
        scratch_types=[
            pltpu.VMEM((3, CHUNK), jnp.int32),
            pltpu.VMEM((CHUNK, D), jnp.float32),
            pltpu.VMEM_SHARED((N, D), jnp.float32),
            pltpu.SemaphoreType.DMA,
        ],
    )
    def k(h_hbm, edata_hbm, out_hbm, edata_v, msgs_v, acc, sem):
        cid = lax.axis_index("c")
        sid = lax.axis_index("s")
        w = sid * NC + cid

        def zero_body(i, carry):
            for j in range(D // LANES):
                msgs_v[i, pl.ds(j * LANES, LANES)] = jnp.zeros(
                    (LANES,), jnp.float32)
            return carry

        lax.fori_loop(0, CHUNK, zero_body, 0)
        for t in range(RITERS):
            rb = sid + t * NS

            @pl.when(rb < NRBLK)
            def _():
                r0 = pl.multiple_of(rb * RBLK, 8)
                pltpu.sync_copy(
                    msgs_v.at[pl.ds(0, RBLK)],
                    acc.at[pl.ds(r0, RBLK)],
                )

        plsc.subcore_barrier()

        def chunk_body(i, carry):
            c = w + i * NW

            @pl.when(c < NCHUNKS)
            def _():
                pltpu.sync_copy(edata_hbm.at[c], edata_v)
                pltpu.async_copy(h_hbm.at[edata_v.at[1]], msgs_v, sem).wait()

                def scale_body(g, carry2):
                    for u in range(4):
                        e = g * 4 + u
                        v = plsc.bitcast(
                            plsc.load_gather(
                                edata_v.at[2],
                                [jnp.full((LANES,), e, jnp.int32)]),
                            jnp.float32)
                        for j in range(D // LANES):
                            sl = msgs_v[e, pl.ds(j * LANES, LANES)]
                            msgs_v[e, pl.ds(j * LANES, LANES)] = sl * v
                    return carry2

                lax.fori_loop(0, CHUNK // 4, scale_body, 0)
                pltpu.sync_copy(msgs_v, acc.at[edata_v.at[0]], add=True)

            return carry

        lax.fori_loop(0, ITERS, chunk_body, 0)
        plsc.subcore_barrier()

        for t in range(RITERS):
            rb = sid + t * NS

            @pl.when(rb < NRBLK)
            def _():
                r0 = pl.multiple_of(rb * RBLK, 8)
                pltpu.sync_copy(
                    acc.at[pl.ds(r0, RBLK)],
                    out_hbm.at[cid, pl.ds(r0, RBLK)],
                )

    return k(h, edata)


def kernel(x, adj_indices, adj_values, W, b):
    h = _matmul_bias(x, W, b)
    rows2 = adj_indices[0].reshape(NCHUNKS, CHUNK)
    cols2 = adj_indices[1].reshape(NCHUNKS, CHUNK)
    vals2 = lax.bitcast_convert_type(adj_values, jnp.int32).reshape(
        NCHUNKS, CHUNK)
    edata = jnp.stack([rows2, cols2, vals2], axis=1)
    parts = _sc_scatter(h, edata)
    return _add2(parts[0], parts[1])
